# Initial kernel scaffold; baseline (speedup 1.0000x reference)
import jax
import jax.numpy as jnp
from jax.experimental import pallas as pl


def kernel(x, edge_index, batch, params):
    raise NotImplementedError("write your pallas kernel here")


# speedup vs baseline: 9.9000x; 9.9000x over previous
import functools

import jax
import jax.numpy as jnp
from jax import lax
from jax.experimental import pallas as pl
from jax.experimental.pallas import tpu as pltpu
from jax.experimental.pallas import tpu_sc as plsc

N = 10000
E = 320000
G = 16
NP = 10240
NW = 32
CH = 128
EPW = E // NW
NCH = (EPW + CH - 1) // CH
EPW_PAD = NCH * CH
RPT = NP // 16

RB2 = 256
NB2 = NP // RB2
RB = 512
NBD = NP // RB
CC = 512
NCC = NP // CC

_f32 = jnp.float32
_i32 = jnp.int32


@functools.lru_cache(maxsize=None)
def _make_seg_sum(F):
    mesh = plsc.VectorSubcoreMesh(core_axis_name="c", subcore_axis_name="s")

    @functools.partial(
        pl.kernel,
        mesh=mesh,
        compiler_params=pltpu.CompilerParams(use_tc_tiling_on_sc=False),
        out_type=jax.ShapeDtypeStruct((2 * NP, F), _f32),
        scratch_types=[
            pltpu.VMEM((EPW_PAD,), _i32),
            pltpu.VMEM((NCH, CH), _i32),
            pltpu.VMEM((CH, F), _f32),
            pltpu.VMEM_SHARED((NP, F), _f32),
            pltpu.SemaphoreType.DMA,
        ],
    )
    def seg_sum(h_hbm, src_hbm, dst_hbm, zero_hbm, out_hbm,
                src_v, dst_v, rows_v, acc_sh, sem):
        c = lax.axis_index("c")
        s = lax.axis_index("s")
        wid = s * 2 + c
        pltpu.sync_copy(zero_hbm.at[pl.ds(s * RPT, RPT)],
                        acc_sh.at[pl.ds(s * RPT, RPT)])
        plsc.subcore_barrier()
        pltpu.sync_copy(src_hbm.at[wid], src_v)
        pltpu.sync_copy(dst_hbm.at[wid], dst_v)

        def body(j, carry):
            pltpu.async_copy(h_hbm.at[src_v.at[pl.ds(j * CH, CH)]],
                             rows_v, sem).wait()
            pltpu.sync_copy(rows_v, acc_sh.at[dst_v.at[j]], add=True)
            return carry

        lax.fori_loop(0, NCH, body, 0)
        plsc.subcore_barrier()
        pltpu.sync_copy(acc_sh.at[pl.ds(s * RPT, RPT)],
                        out_hbm.at[pl.ds(c * NP + s * RPT, RPT)])

    return seg_sum


def _mm_stats(h, agg, w1, b1, mask):
    F = h.shape[1]
    H = w1.shape[1]

    def body(h_ref, a0_ref, a1_ref, w_ref, b_ref, m_ref,
             z_ref, ps_ref, pq_ref, pc_ref):
        z = h_ref[...] + a0_ref[...] + a1_ref[...]
        z1 = jnp.dot(z, w_ref[...], preferred_element_type=_f32) + b_ref[...]
        z_ref[...] = z1
        w = m_ref[...]
        zw = z1 * w
        ps_ref[...] = jnp.sum(zw, axis=0, keepdims=True)[None]
        pq_ref[...] = jnp.sum(z1 * zw, axis=0, keepdims=True)[None]
        pc_ref[...] = jnp.sum(w, axis=0, keepdims=True)[None]

    return pl.pallas_call(
        body,
        grid=(NB2,),
        in_specs=[
            pl.BlockSpec((RB2, F), lambda i: (i, 0)),
            pl.BlockSpec((RB2, F), lambda i: (i, 0)),
            pl.BlockSpec((RB2, F), lambda i: (i + NB2, 0)),
            pl.BlockSpec((F, H), lambda i: (0, 0)),
            pl.BlockSpec((1, H), lambda i: (0, 0)),
            pl.BlockSpec((RB2, 1), lambda i: (i, 0)),
        ],
        out_specs=[
            pl.BlockSpec((RB2, H), lambda i: (i, 0)),
            pl.BlockSpec((1, 1, H), lambda i: (i, 0, 0)),
            pl.BlockSpec((1, 1, H), lambda i: (i, 0, 0)),
            pl.BlockSpec((1, 1, 1), lambda i: (i, 0, 0)),
        ],
        out_shape=[
            jax.ShapeDtypeStruct((NP, H), _f32),
            jax.ShapeDtypeStruct((NB2, 1, H), _f32),
            jax.ShapeDtypeStruct((NB2, 1, H), _f32),
            jax.ShapeDtypeStruct((NB2, 1, 1), _f32),
        ],
    )(h, agg, agg, w1, b1, mask)


def _bn_mm_stats(z1, ps, pq, pc, g1, be1, w2, b2, mask):
    H1 = z1.shape[1]
    H2 = w2.shape[1]

    def body(z_ref, ps_ref, pq_ref, pc_ref, g_ref, be_ref, w_ref, b_ref,
             m_ref, z2_ref, ps2_ref, pq2_ref):
        cnt = jnp.sum(pc_ref[...])
        m = jnp.sum(ps_ref[...], axis=0) / cnt
        v = jnp.sum(pq_ref[...], axis=0) / cnt - m * m
        z = z_ref[...]
        r = jnp.maximum(
            g_ref[...] * (z - m) / jnp.sqrt(v + 1e-5) + be_ref[...], 0.0)
        z2 = jnp.dot(r, w_ref[...], preferred_element_type=_f32) + b_ref[...]
        z2_ref[...] = z2
        w = m_ref[...]
        zw = z2 * w
        ps2_ref[...] = jnp.sum(zw, axis=0, keepdims=True)[None]
        pq2_ref[...] = jnp.sum(z2 * zw, axis=0, keepdims=True)[None]

    return pl.pallas_call(
        body,
        grid=(NB2,),
        in_specs=[
            pl.BlockSpec((RB2, H1), lambda i: (i, 0)),
            pl.BlockSpec((NB2, 1, H1), lambda i: (0, 0, 0)),
            pl.BlockSpec((NB2, 1, H1), lambda i: (0, 0, 0)),
            pl.BlockSpec((NB2, 1, 1), lambda i: (0, 0, 0)),
            pl.BlockSpec((1, H1), lambda i: (0, 0)),
            pl.BlockSpec((1, H1), lambda i: (0, 0)),
            pl.BlockSpec((H1, H2), lambda i: (0, 0)),
            pl.BlockSpec((1, H2), lambda i: (0, 0)),
            pl.BlockSpec((RB2, 1), lambda i: (i, 0)),
        ],
        out_specs=[
            pl.BlockSpec((RB2, H2), lambda i: (i, 0)),
            pl.BlockSpec((1, 1, H2), lambda i: (i, 0, 0)),
            pl.BlockSpec((1, 1, H2), lambda i: (i, 0, 0)),
        ],
        out_shape=[
            jax.ShapeDtypeStruct((NP, H2), _f32),
            jax.ShapeDtypeStruct((NB2, 1, H2), _f32),
            jax.ShapeDtypeStruct((NB2, 1, H2), _f32),
        ],
    )(z1, ps, pq, pc, g1, be1, w2, b2, mask)


def _bn_score(z2, ps, pq, pc, g2, be2, pvec):
    H = z2.shape[1]

    def body(z_ref, ps_ref, pq_ref, pc_ref, g_ref, be_ref, p_ref,
             h_ref, s_ref):
        cnt = jnp.sum(pc_ref[...])
        m = jnp.sum(ps_ref[...], axis=0) / cnt
        v = jnp.sum(pq_ref[...], axis=0) / cnt - m * m
        z = z_ref[...]
        r = jnp.maximum(
            g_ref[...] * (z - m) / jnp.sqrt(v + 1e-5) + be_ref[...], 0.0)
        h_ref[...] = r
        p = p_ref[...]
        nrm = jnp.sqrt(jnp.sum(p * p))
        s_ref[...] = jnp.tanh(
            jnp.dot(r, p.reshape(H, 1), preferred_element_type=_f32) / nrm)

    return pl.pallas_call(
        body,
        grid=(NB2,),
        in_specs=[
            pl.BlockSpec((RB2, H), lambda i: (i, 0)),
            pl.BlockSpec((NB2, 1, H), lambda i: (0, 0, 0)),
            pl.BlockSpec((NB2, 1, H), lambda i: (0, 0, 0)),
            pl.BlockSpec((NB2, 1, 1), lambda i: (0, 0, 0)),
            pl.BlockSpec((1, H), lambda i: (0, 0)),
            pl.BlockSpec((1, H), lambda i: (0, 0)),
            pl.BlockSpec((1, H), lambda i: (0, 0)),
        ],
        out_specs=[
            pl.BlockSpec((RB2, H), lambda i: (i, 0)),
            pl.BlockSpec((RB2, 1), lambda i: (i, 0)),
        ],
        out_shape=[
            jax.ShapeDtypeStruct((NP, H), _f32),
            jax.ShapeDtypeStruct((NP, 1), _f32),
        ],
    )(z2, ps, pq, pc, g2, be2, pvec)


def _rank_pool(h2, s, b_row, mask, s_col, b_col, m_col, c0, c1):
    H = h2.shape[1]

    def body(h_ref, s_ref, b_ref, m_ref, sc_ref, bc_ref, mc_ref,
             c0_ref, c1_ref, keep_ref, hn_ref, pmax_ref, psum_ref, pcnt_ref):
        i = pl.program_id(0)
        srow = s_ref[...]
        brow = b_ref[...]
        mrow = m_ref[...]
        rowid = i * RB + lax.broadcasted_iota(_i32, (RB, 1), 0)

        def cbody(cc, carry):
            rank, n = carry
            sc = sc_ref[:, pl.ds(cc * CC, CC)]
            bc = bc_ref[:, pl.ds(cc * CC, CC)]
            mc = mc_ref[:, pl.ds(cc * CC, CC)]
            colid = cc * CC + lax.broadcasted_iota(_i32, (1, CC), 1)
            same = (bc == brow) & (mc > 0.0)
            beat = (sc > srow) | ((sc == srow) & (colid < rowid))
            rank = rank + jnp.sum((same & beat).astype(_i32), axis=1,
                                  keepdims=True)
            n = n + jnp.sum(same.astype(_i32), axis=1, keepdims=True)
            return rank, n

        rank, n = lax.fori_loop(
            c0_ref[0, 0, 0], c1_ref[0, 0, 0], cbody,
            (jnp.zeros((RB, 1), _i32), jnp.zeros((RB, 1), _i32)))
        k = (4 * n + 4) // 5
        keepb = (mrow > 0.0) & (rank < k)
        keep_ref[...] = keepb.astype(_f32)
        hn = jnp.where(keepb, h_ref[...] * srow, 0.0)
        if H < 128:
            hn_ref[...] = jnp.concatenate(
                [hn, jnp.zeros((RB, 128 - H), _f32)], axis=1)
        else:
            hn_ref[...] = hn
        mxs, sms, cns = [], [], []
        for g in range(G):
            mg = keepb & (brow == g)
            mxs.append(jnp.max(jnp.where(mg, hn, -jnp.inf), axis=0))
            sms.append(jnp.sum(jnp.where(mg, hn, 0.0), axis=0))
            cns.append(jnp.sum(mg.astype(_f32), axis=0))
        pmax_ref[...] = jnp.stack(mxs)[None]
        psum_ref[...] = jnp.stack(sms)[None]
        pcnt_ref[...] = jnp.concatenate(cns).reshape(1, 1, G)

    return pl.pallas_call(
        body,
        grid=(NBD,),
        in_specs=[
            pl.BlockSpec((RB, H), lambda i: (i, 0)),
            pl.BlockSpec((RB, 1), lambda i: (i, 0)),
            pl.BlockSpec((RB, 1), lambda i: (i, 0)),
            pl.BlockSpec((RB, 1), lambda i: (i, 0)),
            pl.BlockSpec((1, NP), lambda i: (0, 0)),
            pl.BlockSpec((1, NP), lambda i: (0, 0)),
            pl.BlockSpec((1, NP), lambda i: (0, 0)),
            pl.BlockSpec((1, 1, 1), lambda i: (i, 0, 0)),
            pl.BlockSpec((1, 1, 1), lambda i: (i, 0, 0)),
        ],
        out_specs=[
            pl.BlockSpec((RB, 1), lambda i: (i, 0)),
            pl.BlockSpec((RB, 128), lambda i: (i, 0)),
            pl.BlockSpec((1, G, H), lambda i: (i, 0, 0)),
            pl.BlockSpec((1, G, H), lambda i: (i, 0, 0)),
            pl.BlockSpec((1, 1, G), lambda i: (i, 0, 0)),
        ],
        out_shape=[
            jax.ShapeDtypeStruct((NP, 1), _f32),
            jax.ShapeDtypeStruct((NP, 128), _f32),
            jax.ShapeDtypeStruct((NBD, G, H), _f32),
            jax.ShapeDtypeStruct((NBD, G, H), _f32),
            jax.ShapeDtypeStruct((NBD, 1, G), _f32),
        ],
    )(h2, s, b_row, mask, s_col, b_col, m_col, c0, c1)


def _decoder(pools, dp):
    def body(*refs):
        (pm1, psm1, pc1, pm2, psm2, pc2, pm3, psm3, pc3,
         w31, b31, g31, be31, w32, b32, g32, be32,
         w21, b21, g21, be21, w22, b22, g22, be22,
         d1w, d1b, out_ref) = refs

        def pool(pm_ref, ps_ref, pc_ref):
            pm = pm_ref[...]
            psv = ps_ref[...]
            pcv = pc_ref[...]
            mx = pm[0]
            sm = psv[0]
            c = pcv[0, 0]
            for j in range(1, NBD):
                mx = jnp.maximum(mx, pm[j])
                sm = sm + psv[j]
                c = c + pcv[j, 0]
            mn = sm / jnp.maximum(c[:, None], 1.0)
            return jnp.concatenate([mx, mn], axis=1)

        x1 = pool(pm1, psm1, pc1)
        x2 = pool(pm2, psm2, pc2)
        x3 = pool(pm3, psm3, pc3)

        def bn16(h, gg, bb):
            m = jnp.sum(h, axis=0, keepdims=True) / float(G)
            v = jnp.sum((h - m) ** 2, axis=0, keepdims=True) / float(G)
            return gg[...] * (h - m) / jnp.sqrt(v + 1e-5) + bb[...]

        def mlp(h, w1r, b1r, g1r, be1r, w2r, b2r, g2r, be2r):
            h = jnp.dot(h, w1r[...], preferred_element_type=_f32) + b1r[...]
            h = jnp.maximum(bn16(h, g1r, be1r), 0.0)
            h = jnp.dot(h, w2r[...], preferred_element_type=_f32) + b2r[...]
            h = jnp.maximum(bn16(h, g2r, be2r), 0.0)
            return h

        xd3 = mlp(x3, w31, b31, g31, be31, w32, b32, g32, be32)
        xd2 = mlp(xd3 + x2, w21, b21, g21, be21, w22, b22, g22, be22)
        logits = (jnp.dot(xd2 + x1, d1w[...], preferred_element_type=_f32)
                  + d1b[...])
        lmax = jnp.max(logits, axis=1, keepdims=True)
        sh = logits - lmax
        out_ref[...] = sh - jnp.log(jnp.sum(jnp.exp(sh), axis=1,
                                            keepdims=True))

    args = list(pools) + list(dp)
    return pl.pallas_call(
        body,
        out_shape=jax.ShapeDtypeStruct((G, 10), _f32),
    )(*args)


def _stage(h, agg, mask, mp, pvec, b_row, b_col, m_col_args):
    c0, c1 = m_col_args
    w1 = mp['W1']
    if w1.shape[0] < 128:
        w1 = jnp.concatenate(
            [w1, jnp.zeros((128 - w1.shape[0], w1.shape[1]), _f32)], axis=0)
    z1, ps, pq, pc = _mm_stats(h, agg, w1, mp['b1'].reshape(1, -1),
                               mask)
    z2, ps2, pq2 = _bn_mm_stats(z1, ps, pq, pc, mp['g1'].reshape(1, -1),
                                mp['be1'].reshape(1, -1), mp['W2'],
                                mp['b2'].reshape(1, -1), mask)
    h2, s = _bn_score(z2, ps2, pq2, pc, mp['g2'].reshape(1, -1),
                      mp['be2'].reshape(1, -1), pvec.reshape(1, -1))
    s_col = s.reshape(1, NP)
    m_col = mask.reshape(1, NP)
    keep, hn, pmax, psum, pcnt = _rank_pool(h2, s, b_row, mask, s_col,
                                            b_col, m_col, c0, c1)
    return hn, keep, (pmax, psum, pcnt)


def kernel(x, edge_index, batch, params):
    src = edge_index[0].astype(_i32)
    dst = edge_index[1].astype(_i32)
    b32 = batch.astype(_i32)

    xp = jnp.concatenate([x, jnp.zeros((NP - N, x.shape[1]), _f32)], axis=0)
    bpad = jnp.concatenate([b32, jnp.full((NP - N,), G, _i32)])
    b_row = bpad.reshape(NP, 1)
    b_col = bpad.reshape(1, NP)
    mask0 = (jnp.arange(NP) < N).astype(_f32).reshape(NP, 1)

    src_w = jnp.concatenate(
        [src.reshape(NW, EPW),
         jnp.zeros((NW, EPW_PAD - EPW), _i32)], axis=1)
    dst_w = jnp.concatenate(
        [dst.reshape(NW, EPW),
         jnp.full((NW, EPW_PAD - EPW), N, _i32)], axis=1).reshape(NW, NCH, CH)

    rows0 = jnp.arange(NBD) * RB
    last = jnp.minimum(rows0 + RB, N) - 1
    blo = bpad[rows0]
    bhi = bpad[last]
    col_lo = jnp.searchsorted(b32, blo, side='left').astype(_i32)
    col_hi = jnp.searchsorted(b32, bhi, side='right').astype(_i32)
    c0 = (col_lo // CC).reshape(NBD, 1, 1)
    c1 = ((col_hi + CC - 1) // CC).reshape(NBD, 1, 1)
    cargs = (c0, c1)

    h = xp
    mask = mask0
    pools = []
    seg = _make_seg_sum(128)
    zinit = jnp.zeros((NP, 128), _f32)
    for mp, pv in ((params['c1'], params['p1']),
                   (params['c2'], params['p2']),
                   (params['c3'], params['p3'])):
        agg = seg(h, src_w, dst_w, zinit)
        h, mask, pool = _stage(h, agg, mask, mp, pv, b_row, b_col, cargs)
        pools.append(pool)

    d3, d2 = params['d3'], params['d2']
    dp = (d3['W1'], d3['b1'].reshape(1, -1), d3['g1'].reshape(1, -1),
          d3['be1'].reshape(1, -1), d3['W2'], d3['b2'].reshape(1, -1),
          d3['g2'].reshape(1, -1), d3['be2'].reshape(1, -1),
          d2['W1'], d2['b1'].reshape(1, -1), d2['g1'].reshape(1, -1),
          d2['be1'].reshape(1, -1), d2['W2'], d2['b2'].reshape(1, -1),
          d2['g2'].reshape(1, -1), d2['be2'].reshape(1, -1),
          params['d1W'], params['d1b'].reshape(1, -1))
    flat_pools = [a for p in pools for a in p]
    return _decoder(flat_pools, dp)
